# baseline (speedup 1.0000x reference)
import jax
import jax.numpy as jnp
from jax import lax
from jax.experimental import pallas as pl
from jax.experimental.pallas import tpu as pltpu
from jax.experimental.pallas import tpu_sc as plsc

_NUM_CORES = 2
_NUM_SUBCORES = 16
_NUM_WORKERS = _NUM_CORES * _NUM_SUBCORES
_CHUNK = 400
_LANES = 16


def kernel(indices, table):
    B, S = indices.shape
    V, D = table.shape
    N = B * S
    per_w = N // _NUM_WORKERS
    flat_idx = indices.reshape(N).astype(jnp.int32)
    mesh = plsc.VectorSubcoreMesh(core_axis_name="c", subcore_axis_name="s")

    @pl.kernel(
        out_type=jax.ShapeDtypeStruct((N, D), table.dtype),
        mesh=mesh,
        scratch_types=[
            pltpu.VMEM((_CHUNK,), jnp.int32),
            pltpu.VMEM((_CHUNK, D), jnp.float32),
            pltpu.SemaphoreType.DMA,
        ],
        compiler_params=pltpu.CompilerParams(use_tc_tiling_on_sc=True),
    )
    def gather_kernel(table_hbm, idx_hbm, out_hbm, idx_v, rows_v, sem):
        wid = lax.axis_index("s") * _NUM_CORES + lax.axis_index("c")
        base = wid * per_w

        @pl.loop(0, per_w, step=_CHUNK)
        def _(c0):
            pltpu.sync_copy(idx_hbm.at[pl.ds(base + c0, _CHUNK)], idx_v)

            @pl.loop(0, _CHUNK, step=_LANES)
            def _(k):
                vec = idx_v[pl.ds(k, _LANES)]
                for j in range(_LANES):
                    pltpu.make_async_copy(
                        table_hbm.at[pl.ds(vec[j], 1)],
                        rows_v.at[pl.ds(k + j, 1)],
                        sem,
                    ).start()

            pltpu.make_async_copy(
                table_hbm.at[pl.ds(0, _CHUNK)], rows_v, sem
            ).wait()

            pltpu.sync_copy(rows_v, out_hbm.at[pl.ds(base + c0, _CHUNK)])

    out = gather_kernel(table, flat_idx)
    return out.reshape(B, S, D)


# speedup vs baseline: 1.1406x; 1.1406x over previous
import jax
import jax.numpy as jnp
from jax import lax
from jax.experimental import pallas as pl
from jax.experimental.pallas import tpu as pltpu
from jax.experimental.pallas import tpu_sc as plsc

_NUM_CORES = 2
_NUM_SUBCORES = 16
_NUM_WORKERS = _NUM_CORES * _NUM_SUBCORES
_LANES = 16
_BROWS = 8


def kernel(indices, table):
    B, S = indices.shape
    V, D = table.shape
    N = B * S
    per_w = N // _NUM_WORKERS
    chunk = _BROWS * S
    n_chunks = per_w // chunk
    b_per_w = B // _NUM_WORKERS
    flat_idx = indices.reshape(N).astype(jnp.int32)
    mesh = plsc.VectorSubcoreMesh(core_axis_name="c", subcore_axis_name="s")

    @pl.kernel(
        out_type=jax.ShapeDtypeStruct((B, S, D), table.dtype),
        mesh=mesh,
        scratch_types=[
            pltpu.VMEM((chunk,), jnp.int32),
            pltpu.VMEM((_BROWS, S, D), jnp.float32),
            pltpu.SemaphoreType.DMA,
        ],
    )
    def gather_kernel(table_hbm, idx_hbm, out_hbm, idx_v, rows_v, sem):
        wid = lax.axis_index("s") * _NUM_CORES + lax.axis_index("c")
        base = wid * per_w
        bbase = wid * b_per_w

        @pl.loop(0, n_chunks)
        def _(ci):
            pltpu.sync_copy(
                idx_hbm.at[pl.ds(base + ci * chunk, chunk)], idx_v
            )

            @pl.loop(0, chunk, step=_LANES)
            def _(k):
                vec = idx_v[pl.ds(k, _LANES)]
                for j in range(_LANES):
                    kk = k + j
                    q = (kk * 5243) >> 18
                    r = kk - q * S
                    pltpu.make_async_copy(
                        table_hbm.at[pl.ds(vec[j], 1)],
                        rows_v.at[q, pl.ds(r, 1)],
                        sem,
                    ).start()

            pltpu.make_async_copy(
                out_hbm.at[pl.ds(0, _BROWS)], rows_v, sem
            ).wait()

            pltpu.sync_copy(
                rows_v, out_hbm.at[pl.ds(bbase + ci * _BROWS, _BROWS)]
            )

    return gather_kernel(table, flat_idx)


# speedup vs baseline: 1.1461x; 1.0048x over previous
import jax
import jax.numpy as jnp
from jax import lax
from jax.experimental import pallas as pl
from jax.experimental.pallas import tpu as pltpu
from jax.experimental.pallas import tpu_sc as plsc

_NUM_CORES = 2
_NUM_SUBCORES = 16
_NUM_WORKERS = _NUM_CORES * _NUM_SUBCORES
_LANES = 16
_BROWS = 8


def kernel(indices, table):
    B, S = indices.shape
    V, D = table.shape
    N = B * S
    per_w = N // _NUM_WORKERS
    chunk = _BROWS * S
    n_chunks = per_w // chunk
    b_per_w = B // _NUM_WORKERS
    flat_idx = indices.reshape(N).astype(jnp.int32)
    mesh = plsc.VectorSubcoreMesh(core_axis_name="c", subcore_axis_name="s")

    @pl.kernel(
        out_type=jax.ShapeDtypeStruct((B, S, D), table.dtype),
        mesh=mesh,
        scratch_types=[
            pltpu.VMEM((chunk,), jnp.int32),
            pltpu.VMEM((_BROWS, S, D), jnp.float32),
            pltpu.SemaphoreType.DMA,
        ],
        compiler_params=pltpu.CompilerParams(needs_layout_passes=False),
    )
    def gather_kernel(table_hbm, idx_hbm, out_hbm, idx_v, rows_v, sem):
        wid = lax.axis_index("s") * _NUM_CORES + lax.axis_index("c")
        base = wid * per_w
        bbase = wid * b_per_w

        @pl.loop(0, n_chunks)
        def _(ci):
            pltpu.sync_copy(
                idx_hbm.at[pl.ds(base + ci * chunk, chunk)], idx_v
            )

            @pl.loop(0, chunk, step=_LANES)
            def _(k):
                vec = idx_v[pl.ds(k, _LANES)]
                for j in range(_LANES):
                    kk = k + j
                    q = (kk * 5243) >> 18
                    r = kk - q * S
                    pltpu.make_async_copy(
                        table_hbm.at[pl.ds(vec[j], 1)],
                        rows_v.at[q, pl.ds(r, 1)],
                        sem,
                    ).start()

            pltpu.make_async_copy(
                out_hbm.at[pl.ds(0, _BROWS)], rows_v, sem
            ).wait()

            pltpu.sync_copy(
                rows_v, out_hbm.at[pl.ds(bbase + ci * _BROWS, _BROWS)]
            )

    return gather_kernel(table, flat_idx)


# speedup vs baseline: 1.1733x; 1.0237x over previous
import jax
import jax.numpy as jnp
from jax import lax
from jax.experimental import pallas as pl
from jax.experimental.pallas import tpu as pltpu
from jax.experimental.pallas import tpu_sc as plsc

_NUM_CORES = 2
_NUM_SUBCORES = 16
_NUM_WORKERS = _NUM_CORES * _NUM_SUBCORES
_LANES = 16
_BROWS = 8


def kernel(indices, table):
    B, S = indices.shape
    V, D = table.shape
    N = B * S
    per_w = N // _NUM_WORKERS
    chunk = _BROWS * S
    n_chunks = per_w // chunk
    b_per_w = B // _NUM_WORKERS
    flat_idx = indices.reshape(N).astype(jnp.int32)
    mesh = plsc.VectorSubcoreMesh(core_axis_name="c", subcore_axis_name="s")

    @pl.kernel(
        out_type=jax.ShapeDtypeStruct((B, S, D), table.dtype),
        mesh=mesh,
        scratch_types=[
            pltpu.VMEM((chunk,), jnp.int32),
            pltpu.VMEM((chunk,), jnp.int32),
            pltpu.VMEM((_BROWS, S, D), jnp.float32),
            pltpu.VMEM((_BROWS, S, D), jnp.float32),
            pltpu.SemaphoreType.DMA,
            pltpu.SemaphoreType.DMA,
        ],
    )
    def gather_kernel(
        table_hbm, idx_hbm, out_hbm, idx_v0, idx_v1, rows_v0, rows_v1, sem0, sem1
    ):
        wid = lax.axis_index("s") * _NUM_CORES + lax.axis_index("c")
        base = wid * per_w
        bbase = wid * b_per_w

        def fire(ci, idx_v, rows_v, sem):
            pltpu.sync_copy(
                idx_hbm.at[pl.ds(base + ci * chunk, chunk)], idx_v
            )

            @pl.loop(0, chunk, step=_LANES)
            def _(k):
                vec = idx_v[pl.ds(k, _LANES)]
                for j in range(_LANES):
                    kk = k + j
                    q = (kk * 5243) >> 18
                    r = kk - q * S
                    pltpu.make_async_copy(
                        table_hbm.at[pl.ds(vec[j], 1)],
                        rows_v.at[q, pl.ds(r, 1)],
                        sem,
                    ).start()

        def drain_and_store(ci, rows_v, sem):
            pltpu.make_async_copy(
                out_hbm.at[pl.ds(0, _BROWS)], rows_v, sem
            ).wait()
            pltpu.sync_copy(
                rows_v, out_hbm.at[pl.ds(bbase + ci * _BROWS, _BROWS)]
            )

        fire(0, idx_v0, rows_v0, sem0)

        @pl.loop(0, n_chunks, step=2)
        def _(ci):
            fire(ci + 1, idx_v1, rows_v1, sem1)
            drain_and_store(ci, rows_v0, sem0)

            @pl.when(ci + 2 < n_chunks)
            def _():
                fire(ci + 2, idx_v0, rows_v0, sem0)

            drain_and_store(ci + 1, rows_v1, sem1)

    return gather_kernel(table, flat_idx)
